# baseline (speedup 1.0000x reference)
import jax
import jax.numpy as jnp
from jax.experimental import pallas as pl


def _id_kernel(x_ref, o_ref):
    o_ref[...] = x_ref[...]


def _mlp(t, p):
    t = t @ p["W1"].T + p["b1"]
    m = t.mean(0)
    v = t.var(0)
    t = jax.nn.relu(p["g1"] * (t - m) / jnp.sqrt(v + 1e-5) + p["be1"])
    t = t @ p["W2"].T + p["b2"]
    m = t.mean(0)
    v = t.var(0)
    t = jax.nn.relu(p["g2"] * (t - m) / jnp.sqrt(v + 1e-5) + p["be2"])
    return t


def _geo(p):
    start = p[:, :2]
    end = p[:, 2:4]
    vec = end - start
    dist = jnp.sqrt(jnp.sum(vec * vec, axis=1, keepdims=True) + 1e-12)
    uv = vec / (dist + 1e-8)
    return start, end, dist, vec, uv


def _layer(h, pos, src, dst, edge_attr, lp):
    h_i = h[dst]
    h_j = h[src]
    pos_i = pos[dst]
    pos_j = pos[src]
    s_i, e_i, d_i, v_i, u_i = _geo(pos_i)
    s_j, e_j, d_j, v_j, u_j = _geo(pos_j)
    dot = jnp.clip(jnp.sum(u_i * u_j, axis=1, keepdims=True), -0.999999, 0.999999)
    angle = jnp.arccos(dot) * (180.0 / jnp.pi)
    cross = u_i[:, 0:1] * u_j[:, 1:2] - u_i[:, 1:2] * u_j[:, 0:1]
    is_lordotic = (cross > 0).astype(h.dtype)
    mid_i = (s_i + e_i) * 0.5
    mid_j = (s_j + e_j) * 0.5
    diff = mid_j - mid_i
    spondy = jnp.sum(diff * u_i, axis=1, keepdims=True)
    perp = diff - spondy * u_i
    height = jnp.sqrt(jnp.sum(perp * perp, axis=1, keepdims=True) + 1e-12)
    geo = jnp.concatenate([d_i, u_i, d_j, u_j, angle, is_lordotic, spondy, height], axis=1)
    msg = _mlp(jnp.concatenate([h_i, h_j, edge_attr, geo], axis=1), lp["msg"])
    aggr = jax.ops.segment_sum(msg, dst, num_segments=h.shape[0])
    upd = _mlp(jnp.concatenate([h, aggr], axis=1), lp["upd"])
    return upd


def kernel(x, pos, edge_index, edge_attr, batch, params):
    src = edge_index[0]
    dst = edge_index[1]
    h = x @ params["lin_in"]["W"].T + params["lin_in"]["b"]
    for lp in params["layers"]:
        h = h + _layer(h, pos, src, dst, edge_attr, lp)
    num_graphs = 64
    sums = jax.ops.segment_sum(h, batch, num_segments=num_graphs)
    cnt = jax.ops.segment_sum(jnp.ones((h.shape[0], 1), h.dtype), batch, num_segments=num_graphs)
    h_graph = sums / jnp.maximum(cnt, 1.0)
    out = h_graph @ params["lin_pred"]["W"].T + params["lin_pred"]["b"]
    out = pl.pallas_call(
        _id_kernel, out_shape=jax.ShapeDtypeStruct(out.shape, out.dtype)
    )(out)
    return out.reshape(-1)


# speedup vs baseline: 1.3191x; 1.3191x over previous
import functools
import jax
import jax.numpy as jnp
from jax.experimental import pallas as pl

N = 10000
E = 320000
D = 128
NG = 64
BE = 2560
GRID_E = E // BE

_ACOS_C = (1.5707963050, -0.2145988016, 0.0889789874, -0.0501743046,
           0.0308918810, -0.0170881256, 0.0066700901, -0.0012624911)


def _acos(x):
    y = jnp.abs(x)
    p = _ACOS_C[7]
    for c in (_ACOS_C[6], _ACOS_C[5], _ACOS_C[4], _ACOS_C[3], _ACOS_C[2],
              _ACOS_C[1], _ACOS_C[0]):
        p = p * y + c
    r = jnp.sqrt(jnp.maximum(1.0 - y, 0.0)) * p
    return jnp.where(x < 0.0, jnp.pi - r, r)


def _init_kernel(x_ref, posp_ref, winT_ref, bin_ref, w1aT_ref, w1bT_ref,
                 b1_ref, h_ref, ph_ref, qh_ref, pvec_ref):
    h = jnp.dot(x_ref[...], winT_ref[...],
                preferred_element_type=jnp.float32) + bin_ref[...]
    h_ref[...] = h
    ph_ref[...] = jnp.dot(h, w1aT_ref[...], preferred_element_type=jnp.float32)
    qh_ref[...] = jnp.dot(h, w1bT_ref[...],
                          preferred_element_type=jnp.float32) + b1_ref[...]
    p = posp_ref[...]
    s = p[:, 0:2]
    e = p[:, 2:4]
    vec = e - s
    dist = jnp.sqrt(jnp.sum(vec * vec, axis=1, keepdims=True) + 1e-12)
    uv = vec / (dist + 1e-8)
    mid = (s + e) * 0.5
    z = jnp.zeros_like(p[:, 0:1])
    pvec_ref[...] = jnp.concatenate(
        [dist, uv, mid, z, z, z, z, z, z, z, z, z, z, z], axis=1)


def _node_init(x16, pos16, winT, binb, w1aT, w1bT, b1):
    return pl.pallas_call(
        _init_kernel,
        out_shape=(
            jax.ShapeDtypeStruct((N, D), jnp.float32),
            jax.ShapeDtypeStruct((N, D), jnp.float32),
            jax.ShapeDtypeStruct((N, D), jnp.float32),
            jax.ShapeDtypeStruct((N, 16), jnp.float32),
        ),
    )(x16, pos16, winT, binb, w1aT, w1bT, b1)


def _feat_kernel(pvd_ref, pvs_ref, ea_ref, feat_ref):
    pvd = pvd_ref[...]
    pvs = pvs_ref[...]
    d_i = pvd[:, 0:1]
    u_i = pvd[:, 1:3]
    m_i = pvd[:, 3:5]
    d_j = pvs[:, 0:1]
    u_j = pvs[:, 1:3]
    m_j = pvs[:, 3:5]
    dot = jnp.clip(jnp.sum(u_i * u_j, axis=1, keepdims=True),
                   -0.999999, 0.999999)
    angle = _acos(dot) * (180.0 / jnp.pi)
    cross = u_i[:, 0:1] * u_j[:, 1:2] - u_i[:, 1:2] * u_j[:, 0:1]
    is_lord = (cross > 0.0).astype(jnp.float32)
    diff = m_j - m_i
    spondy = jnp.sum(diff * u_i, axis=1, keepdims=True)
    perp = diff - spondy * u_i
    height = jnp.sqrt(jnp.sum(perp * perp, axis=1, keepdims=True) + 1e-12)
    z = jnp.zeros_like(d_i)
    feat_ref[...] = jnp.concatenate(
        [ea_ref[...][:, 0:4], d_i, u_i, d_j, u_j, angle, is_lord, spondy,
         height, z, z], axis=1)


def _edge_feat(pvd, pvs, ea16):
    spec16 = pl.BlockSpec((BE, 16), lambda i: (i, 0))
    return pl.pallas_call(
        _feat_kernel,
        grid=(GRID_E,),
        in_specs=[spec16, spec16, spec16],
        out_specs=spec16,
        out_shape=jax.ShapeDtypeStruct((E, 16), jnp.float32),
    )(pvd, pvs, ea16)


def _stats1_kernel(ghd_ref, ghs_ref, feat_ref, wcdT_ref, acc_ref):
    t1 = (ghd_ref[...] + ghs_ref[...]
          + jnp.dot(feat_ref[...], wcdT_ref[...],
                    preferred_element_type=jnp.float32))
    i = pl.program_id(0)

    @pl.when(i == 0)
    def _():
        acc_ref[...] = jnp.zeros_like(acc_ref)

    acc_ref[0:1, :] += jnp.sum(t1, axis=0, keepdims=True)
    acc_ref[1:2, :] += jnp.sum(t1 * t1, axis=0, keepdims=True)


def _stats1(ghd, ghs, feat, wcdT):
    specD = pl.BlockSpec((BE, D), lambda i: (i, 0))
    return pl.pallas_call(
        _stats1_kernel,
        grid=(GRID_E,),
        in_specs=[specD, specD, pl.BlockSpec((BE, 16), lambda i: (i, 0)),
                  pl.BlockSpec((16, D), lambda i: (0, 0))],
        out_specs=pl.BlockSpec((8, D), lambda i: (0, 0)),
        out_shape=jax.ShapeDtypeStruct((8, D), jnp.float32),
    )(ghd, ghs, feat, wcdT)


def _t2_kernel(ghd_ref, ghs_ref, feat_ref, wcdT_ref, ac1_ref, w2T_ref,
               t2_ref, acc_ref):
    t1 = (ghd_ref[...] + ghs_ref[...]
          + jnp.dot(feat_ref[...], wcdT_ref[...],
                    preferred_element_type=jnp.float32))
    u = jnp.maximum(ac1_ref[0:1, :] * t1 + ac1_ref[1:2, :], 0.0)
    t2 = jnp.dot(u, w2T_ref[...], preferred_element_type=jnp.float32)
    t2_ref[...] = t2
    i = pl.program_id(0)

    @pl.when(i == 0)
    def _():
        acc_ref[...] = jnp.zeros_like(acc_ref)

    acc_ref[0:1, :] += jnp.sum(t2, axis=0, keepdims=True)
    acc_ref[1:2, :] += jnp.sum(t2 * t2, axis=0, keepdims=True)


def _t2_pass(ghd, ghs, feat, wcdT, ac1, w2T):
    specD = pl.BlockSpec((BE, D), lambda i: (i, 0))
    return pl.pallas_call(
        _t2_kernel,
        grid=(GRID_E,),
        in_specs=[specD, specD, pl.BlockSpec((BE, 16), lambda i: (i, 0)),
                  pl.BlockSpec((16, D), lambda i: (0, 0)),
                  pl.BlockSpec((8, D), lambda i: (0, 0)),
                  pl.BlockSpec((D, D), lambda i: (0, 0))],
        out_specs=(specD, pl.BlockSpec((8, D), lambda i: (0, 0))),
        out_shape=(jax.ShapeDtypeStruct((E, D), jnp.float32),
                   jax.ShapeDtypeStruct((8, D), jnp.float32)),
    )(ghd, ghs, feat, wcdT, ac1, w2T)


def _msg_kernel(t2_ref, ac2_ref, msg_ref):
    msg_ref[...] = jnp.maximum(
        ac2_ref[0:1, :] * t2_ref[...] + ac2_ref[1:2, :], 0.0)


def _msg_pass(t2, ac2):
    specD = pl.BlockSpec((BE, D), lambda i: (i, 0))
    return pl.pallas_call(
        _msg_kernel,
        grid=(GRID_E,),
        in_specs=[specD, pl.BlockSpec((8, D), lambda i: (0, 0))],
        out_specs=specD,
        out_shape=jax.ShapeDtypeStruct((E, D), jnp.float32),
    )(t2, ac2)


def _upd_kernel(h_ref, ag_ref, wu1aT_ref, wu1bT_ref, bu1_ref, g1_ref,
                be1_ref, wu2T_ref, bu2_ref, g2_ref, be2_ref, w1aT_ref,
                w1bT_ref, b1n_ref, hn_ref, ph_ref, qh_ref):
    h = h_ref[...]
    t = (jnp.dot(h, wu1aT_ref[...], preferred_element_type=jnp.float32)
         + jnp.dot(ag_ref[...], wu1bT_ref[...],
                   preferred_element_type=jnp.float32) + bu1_ref[...])
    m = jnp.mean(t, axis=0, keepdims=True)
    v = jnp.mean(t * t, axis=0, keepdims=True) - m * m
    t = jnp.maximum(g1_ref[...] * (t - m) / jnp.sqrt(v + 1e-5)
                    + be1_ref[...], 0.0)
    t = jnp.dot(t, wu2T_ref[...], preferred_element_type=jnp.float32) \
        + bu2_ref[...]
    m = jnp.mean(t, axis=0, keepdims=True)
    v = jnp.mean(t * t, axis=0, keepdims=True) - m * m
    t = jnp.maximum(g2_ref[...] * (t - m) / jnp.sqrt(v + 1e-5)
                    + be2_ref[...], 0.0)
    hn = h + t
    hn_ref[...] = hn
    ph_ref[...] = jnp.dot(hn, w1aT_ref[...],
                          preferred_element_type=jnp.float32)
    qh_ref[...] = jnp.dot(hn, w1bT_ref[...],
                          preferred_element_type=jnp.float32) + b1n_ref[...]


def _node_update(h, aggr, up, w1aT_next, w1bT_next, b1_next):
    return pl.pallas_call(
        _upd_kernel,
        out_shape=(jax.ShapeDtypeStruct((N, D), jnp.float32),
                   jax.ShapeDtypeStruct((N, D), jnp.float32),
                   jax.ShapeDtypeStruct((N, D), jnp.float32)),
    )(h, aggr, up["W1"].T, up["W1b"], bu1 := up["b1"].reshape(1, D),
      up["g1"].reshape(1, D), up["be1"].reshape(1, D), up["W2"].T,
      up["b2"].reshape(1, D), up["g2"].reshape(1, D),
      up["be2"].reshape(1, D), w1aT_next, w1bT_next, b1_next)


def _final_kernel(h_ref, ag_ref, wu1aT_ref, wu1bT_ref, bu1_ref, g1_ref,
                  be1_ref, wu2T_ref, bu2_ref, g2_ref, be2_ref, batch_ref,
                  wpT_ref, bp_ref, out_ref):
    h = h_ref[...]
    t = (jnp.dot(h, wu1aT_ref[...], preferred_element_type=jnp.float32)
         + jnp.dot(ag_ref[...], wu1bT_ref[...],
                   preferred_element_type=jnp.float32) + bu1_ref[...])
    m = jnp.mean(t, axis=0, keepdims=True)
    v = jnp.mean(t * t, axis=0, keepdims=True) - m * m
    t = jnp.maximum(g1_ref[...] * (t - m) / jnp.sqrt(v + 1e-5)
                    + be1_ref[...], 0.0)
    t = jnp.dot(t, wu2T_ref[...], preferred_element_type=jnp.float32) \
        + bu2_ref[...]
    m = jnp.mean(t, axis=0, keepdims=True)
    v = jnp.mean(t * t, axis=0, keepdims=True) - m * m
    t = jnp.maximum(g2_ref[...] * (t - m) / jnp.sqrt(v + 1e-5)
                    + be2_ref[...], 0.0)
    hn = h + t
    b = batch_ref[...][:, 0:1]
    gid = jax.lax.broadcasted_iota(jnp.int32, (1, NG), 1)
    onehot = (b == gid).astype(jnp.float32)
    sums = jax.lax.dot_general(onehot, hn, (((0,), (0,)), ((), ())),
                               preferred_element_type=jnp.float32)
    cnt8 = jax.lax.dot_general(onehot, jnp.ones((h.shape[0], 8), jnp.float32),
                               (((0,), (0,)), ((), ())),
                               preferred_element_type=jnp.float32)
    hg = sums / jnp.maximum(cnt8[:, 0:1], 1.0)
    out_ref[...] = jnp.dot(hg, wpT_ref[...],
                           preferred_element_type=jnp.float32) + bp_ref[...]


def _final(h, aggr, up, batchf, wpT, bp):
    return pl.pallas_call(
        _final_kernel,
        out_shape=jax.ShapeDtypeStruct((NG, 8), jnp.float32),
    )(h, aggr, up["W1"].T, up["W1b"], up["b1"].reshape(1, D),
      up["g1"].reshape(1, D), up["be1"].reshape(1, D), up["W2"].T,
      up["b2"].reshape(1, D), up["g2"].reshape(1, D),
      up["be2"].reshape(1, D), batchf, wpT, bp)


def _affine(stats, g, be, n):
    m = stats[0] / n
    v = stats[1] / n - m * m
    a = g / jnp.sqrt(v + 1e-5)
    c = be - m * a
    return jnp.concatenate([a.reshape(1, D), c.reshape(1, D),
                            jnp.zeros((6, D), jnp.float32)], axis=0)


def kernel(x, pos, edge_index, edge_attr, batch, params):
    src = edge_index[0].astype(jnp.int32)
    dst = edge_index[1].astype(jnp.int32)
    x16 = jnp.pad(x, ((0, 0), (0, 2)))
    pos16 = jnp.pad(pos, ((0, 0), (0, 12)))
    ea16 = jnp.pad(edge_attr, ((0, 0), (0, 12)))
    batchf = jnp.pad(batch.astype(jnp.int32).reshape(N, 1),
                     ((0, 0), (0, 7)))

    lps = params["layers"]
    winT = jnp.pad(params["lin_in"]["W"].T, ((0, 2), (0, 0)))
    l0 = lps[0]["msg"]
    h, ph, qh, pvec = _node_init(
        x16, pos16, winT, params["lin_in"]["b"].reshape(1, D),
        l0["W1"][:, :D].T, l0["W1"][:, D:2 * D].T, l0["b1"].reshape(1, D))

    pvd = pvec[dst]
    pvs = pvec[src]
    feat = _edge_feat(pvd, pvs, ea16)

    for li, lp in enumerate(lps):
        mp = lp["msg"]
        wcdT = jnp.pad(mp["W1"][:, 2 * D:].T, ((0, 2), (0, 0)))
        ghd = ph[dst]
        ghs = qh[src]
        st1 = _stats1(ghd, ghs, feat, wcdT)
        ac1 = _affine(st1, mp["g1"], mp["be1"], float(E))
        t2, st2 = _t2_pass(ghd, ghs, feat, wcdT, ac1, mp["W2"].T)
        s2 = st2[0] + float(E) * mp["b2"]
        q2 = st2[1] + 2.0 * mp["b2"] * st2[0] + float(E) * mp["b2"] ** 2
        m2 = s2 / float(E)
        v2 = q2 / float(E) - m2 * m2
        a2 = mp["g2"] / jnp.sqrt(v2 + 1e-5)
        c2 = mp["be2"] - m2 * a2 + a2 * mp["b2"]
        ac2 = jnp.concatenate([a2.reshape(1, D), c2.reshape(1, D),
                               jnp.zeros((6, D), jnp.float32)], axis=0)
        msg = _msg_pass(t2, ac2)
        aggr = jax.ops.segment_sum(msg, dst, num_segments=N)
        up = dict(lps[li]["upd"])
        up["W1b"] = up["W1"][:, D:].T
        up["W1"] = up["W1"][:, :D]
        if li + 1 < len(lps):
            nx = lps[li + 1]["msg"]
            h, ph, qh = _node_update(h, aggr, up, nx["W1"][:, :D].T,
                                     nx["W1"][:, D:2 * D].T,
                                     nx["b1"].reshape(1, D))
        else:
            wpT = jnp.pad(params["lin_pred"]["W"].T, ((0, 0), (0, 7)))
            bp = jnp.pad(params["lin_pred"]["b"].reshape(1, 1),
                         ((0, 0), (0, 7)))
            out = _final(h, aggr, up, batchf, wpT, bp)
    return out[:, 0].reshape(-1)


# speedup vs baseline: 2.8819x; 2.1847x over previous
import functools
import jax
from jax import lax
import jax.numpy as jnp
from jax.experimental import pallas as pl
from jax.experimental.pallas import tpu as pltpu
from jax.experimental.pallas import tpu_sc as plsc

N = 10000
E = 320000
D = 128
NG = 64
BE = 2560
GRID_E = E // BE
GW = 128
NC = 2
NS = 16
NCHUNK = E // GW
EPAD = 327680

_SC_MESH = plsc.VectorSubcoreMesh(core_axis_name="c", subcore_axis_name="s")


def _sc_gather2(tab_a, tab_b, idx_a2d, idx_b2d, width):
    @functools.partial(
        pl.kernel,
        out_type=(jax.ShapeDtypeStruct((EPAD, width), jnp.float32),
                  jax.ShapeDtypeStruct((EPAD, width), jnp.float32)),
        mesh=_SC_MESH,
    )
    def k(a_hbm, b_hbm, ia_hbm, ib_hbm, oa_hbm, ob_hbm):
        def body(ia_v, ib_v, oa_v, ob_v):
            pltpu.sync_copy(a_hbm.at[ia_v.at[0]], oa_v)
            pltpu.sync_copy(b_hbm.at[ib_v.at[0]], ob_v)

        pltpu.emit_pipeline(
            body,
            grid=(EPAD // GW,),
            in_specs=[pl.BlockSpec((1, GW), lambda i: (0, i)),
                      pl.BlockSpec((1, GW), lambda i: (0, i))],
            out_specs=[pl.BlockSpec((GW, width), lambda i: (i, 0)),
                       pl.BlockSpec((GW, width), lambda i: (i, 0))],
            core_axis_name=("c", "s"),
            dimension_semantics=(pltpu.PARALLEL,),
        )(ia_hbm, ib_hbm, oa_hbm, ob_hbm)

    return k(tab_a, tab_b, idx_a2d, idx_b2d)


def _sc_scatter_add(msg, dst_flat):
    @functools.partial(
        pl.kernel,
        out_type=jax.ShapeDtypeStruct((NC, N, D), jnp.float32),
        mesh=_SC_MESH,
        scratch_types=[
            pltpu.VMEM((GW,), jnp.int32),
            pltpu.VMEM((GW, D), jnp.float32),
            pltpu.VMEM_SHARED((N, D), jnp.float32),
        ],
    )
    def k(msg_hbm, di_hbm, z_hbm, out_hbm, idx_v, rows_v, aggr_sh):
        cid = lax.axis_index("c")
        sid = lax.axis_index("s")

        @pl.loop(0, (N // 80 + NS - 1) // NS)
        def _(kk):
            row0 = (sid + kk * NS) * 80

            @pl.when(row0 < N)
            def _():
                pltpu.sync_copy(z_hbm.at[pl.ds(row0, 80)],
                                aggr_sh.at[pl.ds(row0, 80)])

        plsc.subcore_barrier()

        wid = cid * NS + sid

        @pl.loop(0, (NCHUNK + NC * NS - 1) // (NC * NS))
        def _(kk):
            c = wid + kk * NC * NS

            @pl.when(c < NCHUNK)
            def _():
                off = c * GW
                pltpu.sync_copy(di_hbm.at[pl.ds(off, GW)], idx_v)
                pltpu.sync_copy(msg_hbm.at[pl.ds(off, GW)], rows_v)
                pltpu.sync_copy(rows_v, aggr_sh.at[idx_v], add=True)

        plsc.subcore_barrier()

        @pl.when(sid == 0)
        def _():
            pltpu.sync_copy(aggr_sh, out_hbm.at[cid])

    return k(msg, dst_flat, jnp.zeros((N, D), jnp.float32))

_ACOS_C = (1.5707963050, -0.2145988016, 0.0889789874, -0.0501743046,
           0.0308918810, -0.0170881256, 0.0066700901, -0.0012624911)


def _acos(x):
    y = jnp.abs(x)
    p = _ACOS_C[7]
    for c in (_ACOS_C[6], _ACOS_C[5], _ACOS_C[4], _ACOS_C[3], _ACOS_C[2],
              _ACOS_C[1], _ACOS_C[0]):
        p = p * y + c
    r = jnp.sqrt(jnp.maximum(1.0 - y, 0.0)) * p
    return jnp.where(x < 0.0, jnp.pi - r, r)


def _init_kernel(x_ref, posp_ref, winT_ref, bin_ref, w1aT_ref, w1bT_ref,
                 b1_ref, h_ref, ph_ref, qh_ref, pvec_ref):
    h = jnp.dot(x_ref[...], winT_ref[...],
                preferred_element_type=jnp.float32) + bin_ref[...]
    h_ref[...] = h
    ph_ref[...] = jnp.dot(h, w1aT_ref[...], preferred_element_type=jnp.float32)
    qh_ref[...] = jnp.dot(h, w1bT_ref[...],
                          preferred_element_type=jnp.float32) + b1_ref[...]
    p = posp_ref[...]
    s = p[:, 0:2]
    e = p[:, 2:4]
    vec = e - s
    dist = jnp.sqrt(jnp.sum(vec * vec, axis=1, keepdims=True) + 1e-12)
    uv = vec / (dist + 1e-8)
    mid = (s + e) * 0.5
    z = jnp.zeros((p.shape[0], 123), jnp.float32)
    pvec_ref[...] = jnp.concatenate([dist, uv, mid, z], axis=1)


def _node_init(x16, pos16, winT, binb, w1aT, w1bT, b1):
    return pl.pallas_call(
        _init_kernel,
        out_shape=(
            jax.ShapeDtypeStruct((N, D), jnp.float32),
            jax.ShapeDtypeStruct((N, D), jnp.float32),
            jax.ShapeDtypeStruct((N, D), jnp.float32),
            jax.ShapeDtypeStruct((N, D), jnp.float32),
        ),
    )(x16, pos16, winT, binb, w1aT, w1bT, b1)


def _feat_kernel(pvd_ref, pvs_ref, ea_ref, feat_ref):
    pvd = pvd_ref[...]
    pvs = pvs_ref[...]
    d_i = pvd[:, 0:1]
    u_i = pvd[:, 1:3]
    m_i = pvd[:, 3:5]
    d_j = pvs[:, 0:1]
    u_j = pvs[:, 1:3]
    m_j = pvs[:, 3:5]
    dot = jnp.clip(jnp.sum(u_i * u_j, axis=1, keepdims=True),
                   -0.999999, 0.999999)
    angle = _acos(dot) * (180.0 / jnp.pi)
    cross = u_i[:, 0:1] * u_j[:, 1:2] - u_i[:, 1:2] * u_j[:, 0:1]
    is_lord = (cross > 0.0).astype(jnp.float32)
    diff = m_j - m_i
    spondy = jnp.sum(diff * u_i, axis=1, keepdims=True)
    perp = diff - spondy * u_i
    height = jnp.sqrt(jnp.sum(perp * perp, axis=1, keepdims=True) + 1e-12)
    z = jnp.zeros_like(d_i)
    feat_ref[...] = jnp.concatenate(
        [ea_ref[...][:, 0:4], d_i, u_i, d_j, u_j, angle, is_lord, spondy,
         height, z, z], axis=1)


def _edge_feat(pvd, pvs, ea16):
    spec16 = pl.BlockSpec((BE, 16), lambda i: (i, 0))
    specD = pl.BlockSpec((BE, D), lambda i: (i, 0))
    return pl.pallas_call(
        _feat_kernel,
        grid=(GRID_E,),
        in_specs=[specD, specD, spec16],
        out_specs=spec16,
        out_shape=jax.ShapeDtypeStruct((E, 16), jnp.float32),
    )(pvd, pvs, ea16)


def _stats1_kernel(ghd_ref, ghs_ref, feat_ref, wcdT_ref, acc_ref):
    t1 = (ghd_ref[...] + ghs_ref[...]
          + jnp.dot(feat_ref[...], wcdT_ref[...],
                    preferred_element_type=jnp.float32))
    i = pl.program_id(0)

    @pl.when(i == 0)
    def _():
        acc_ref[...] = jnp.zeros_like(acc_ref)

    acc_ref[0:1, :] += jnp.sum(t1, axis=0, keepdims=True)
    acc_ref[1:2, :] += jnp.sum(t1 * t1, axis=0, keepdims=True)


def _stats1(ghd, ghs, feat, wcdT):
    specD = pl.BlockSpec((BE, D), lambda i: (i, 0))
    return pl.pallas_call(
        _stats1_kernel,
        grid=(GRID_E,),
        in_specs=[specD, specD, pl.BlockSpec((BE, 16), lambda i: (i, 0)),
                  pl.BlockSpec((16, D), lambda i: (0, 0))],
        out_specs=pl.BlockSpec((8, D), lambda i: (0, 0)),
        out_shape=jax.ShapeDtypeStruct((8, D), jnp.float32),
    )(ghd, ghs, feat, wcdT)


def _t2_kernel(ghd_ref, ghs_ref, feat_ref, wcdT_ref, ac1_ref, w2T_ref,
               t2_ref, acc_ref):
    t1 = (ghd_ref[...] + ghs_ref[...]
          + jnp.dot(feat_ref[...], wcdT_ref[...],
                    preferred_element_type=jnp.float32))
    u = jnp.maximum(ac1_ref[0:1, :] * t1 + ac1_ref[1:2, :], 0.0)
    t2 = jnp.dot(u, w2T_ref[...], preferred_element_type=jnp.float32)
    t2_ref[...] = t2
    i = pl.program_id(0)

    @pl.when(i == 0)
    def _():
        acc_ref[...] = jnp.zeros_like(acc_ref)

    acc_ref[0:1, :] += jnp.sum(t2, axis=0, keepdims=True)
    acc_ref[1:2, :] += jnp.sum(t2 * t2, axis=0, keepdims=True)


def _t2_pass(ghd, ghs, feat, wcdT, ac1, w2T):
    specD = pl.BlockSpec((BE, D), lambda i: (i, 0))
    return pl.pallas_call(
        _t2_kernel,
        grid=(GRID_E,),
        in_specs=[specD, specD, pl.BlockSpec((BE, 16), lambda i: (i, 0)),
                  pl.BlockSpec((16, D), lambda i: (0, 0)),
                  pl.BlockSpec((8, D), lambda i: (0, 0)),
                  pl.BlockSpec((D, D), lambda i: (0, 0))],
        out_specs=(specD, pl.BlockSpec((8, D), lambda i: (0, 0))),
        out_shape=(jax.ShapeDtypeStruct((E, D), jnp.float32),
                   jax.ShapeDtypeStruct((8, D), jnp.float32)),
    )(ghd, ghs, feat, wcdT, ac1, w2T)


def _msg_kernel(t2_ref, ac2_ref, msg_ref):
    msg_ref[...] = jnp.maximum(
        ac2_ref[0:1, :] * t2_ref[...] + ac2_ref[1:2, :], 0.0)


def _msg_pass(t2, ac2):
    specD = pl.BlockSpec((BE, D), lambda i: (i, 0))
    return pl.pallas_call(
        _msg_kernel,
        grid=(GRID_E,),
        in_specs=[specD, pl.BlockSpec((8, D), lambda i: (0, 0))],
        out_specs=specD,
        out_shape=jax.ShapeDtypeStruct((E, D), jnp.float32),
    )(t2, ac2)


def _upd_kernel(h_ref, ag_ref, agb_ref, wu1aT_ref, wu1bT_ref, bu1_ref,
                g1_ref, be1_ref, wu2T_ref, bu2_ref, g2_ref, be2_ref,
                w1aT_ref, w1bT_ref, b1n_ref, hn_ref, ph_ref, qh_ref):
    h = h_ref[...]
    ag = ag_ref[...] + agb_ref[...]
    t = (jnp.dot(h, wu1aT_ref[...], preferred_element_type=jnp.float32)
         + jnp.dot(ag, wu1bT_ref[...],
                   preferred_element_type=jnp.float32) + bu1_ref[...])
    m = jnp.mean(t, axis=0, keepdims=True)
    v = jnp.mean(t * t, axis=0, keepdims=True) - m * m
    t = jnp.maximum(g1_ref[...] * (t - m) / jnp.sqrt(v + 1e-5)
                    + be1_ref[...], 0.0)
    t = jnp.dot(t, wu2T_ref[...], preferred_element_type=jnp.float32) \
        + bu2_ref[...]
    m = jnp.mean(t, axis=0, keepdims=True)
    v = jnp.mean(t * t, axis=0, keepdims=True) - m * m
    t = jnp.maximum(g2_ref[...] * (t - m) / jnp.sqrt(v + 1e-5)
                    + be2_ref[...], 0.0)
    hn = h + t
    hn_ref[...] = hn
    ph_ref[...] = jnp.dot(hn, w1aT_ref[...],
                          preferred_element_type=jnp.float32)
    qh_ref[...] = jnp.dot(hn, w1bT_ref[...],
                          preferred_element_type=jnp.float32) + b1n_ref[...]


def _node_update(h, aggr, aggr_b, up, w1aT_next, w1bT_next, b1_next):
    return pl.pallas_call(
        _upd_kernel,
        out_shape=(jax.ShapeDtypeStruct((N, D), jnp.float32),
                   jax.ShapeDtypeStruct((N, D), jnp.float32),
                   jax.ShapeDtypeStruct((N, D), jnp.float32)),
    )(h, aggr, aggr_b, up["W1"].T, up["W1b"], bu1 := up["b1"].reshape(1, D),
      up["g1"].reshape(1, D), up["be1"].reshape(1, D), up["W2"].T,
      up["b2"].reshape(1, D), up["g2"].reshape(1, D),
      up["be2"].reshape(1, D), w1aT_next, w1bT_next, b1_next)


def _final_kernel(h_ref, ag_ref, agb_ref, wu1aT_ref, wu1bT_ref, bu1_ref,
                  g1_ref, be1_ref, wu2T_ref, bu2_ref, g2_ref, be2_ref,
                  batch_ref, wpT_ref, bp_ref, out_ref):
    h = h_ref[...]
    ag = ag_ref[...] + agb_ref[...]
    t = (jnp.dot(h, wu1aT_ref[...], preferred_element_type=jnp.float32)
         + jnp.dot(ag, wu1bT_ref[...],
                   preferred_element_type=jnp.float32) + bu1_ref[...])
    m = jnp.mean(t, axis=0, keepdims=True)
    v = jnp.mean(t * t, axis=0, keepdims=True) - m * m
    t = jnp.maximum(g1_ref[...] * (t - m) / jnp.sqrt(v + 1e-5)
                    + be1_ref[...], 0.0)
    t = jnp.dot(t, wu2T_ref[...], preferred_element_type=jnp.float32) \
        + bu2_ref[...]
    m = jnp.mean(t, axis=0, keepdims=True)
    v = jnp.mean(t * t, axis=0, keepdims=True) - m * m
    t = jnp.maximum(g2_ref[...] * (t - m) / jnp.sqrt(v + 1e-5)
                    + be2_ref[...], 0.0)
    hn = h + t
    b = batch_ref[...][:, 0:1]
    gid = jax.lax.broadcasted_iota(jnp.int32, (1, NG), 1)
    onehot = (b == gid).astype(jnp.float32)
    sums = jax.lax.dot_general(onehot, hn, (((0,), (0,)), ((), ())),
                               preferred_element_type=jnp.float32)
    cnt8 = jax.lax.dot_general(onehot, jnp.ones((h.shape[0], 8), jnp.float32),
                               (((0,), (0,)), ((), ())),
                               preferred_element_type=jnp.float32)
    hg = sums / jnp.maximum(cnt8[:, 0:1], 1.0)
    out_ref[...] = jnp.dot(hg, wpT_ref[...],
                           preferred_element_type=jnp.float32) + bp_ref[...]


def _final(h, aggr, aggr_b, up, batchf, wpT, bp):
    return pl.pallas_call(
        _final_kernel,
        out_shape=jax.ShapeDtypeStruct((NG, 8), jnp.float32),
    )(h, aggr, aggr_b, up["W1"].T, up["W1b"], up["b1"].reshape(1, D),
      up["g1"].reshape(1, D), up["be1"].reshape(1, D), up["W2"].T,
      up["b2"].reshape(1, D), up["g2"].reshape(1, D),
      up["be2"].reshape(1, D), batchf, wpT, bp)


def _affine(stats, g, be, n):
    m = stats[0] / n
    v = stats[1] / n - m * m
    a = g / jnp.sqrt(v + 1e-5)
    c = be - m * a
    return jnp.concatenate([a.reshape(1, D), c.reshape(1, D),
                            jnp.zeros((6, D), jnp.float32)], axis=0)


def kernel(x, pos, edge_index, edge_attr, batch, params):
    src = edge_index[0].astype(jnp.int32)
    dst = edge_index[1].astype(jnp.int32)
    x16 = jnp.pad(x, ((0, 0), (0, 2)))
    pos16 = jnp.pad(pos, ((0, 0), (0, 12)))
    ea16 = jnp.pad(edge_attr, ((0, 0), (0, 12)))
    batchf = jnp.pad(batch.astype(jnp.int32).reshape(N, 1),
                     ((0, 0), (0, 7)))

    lps = params["layers"]
    winT = jnp.pad(params["lin_in"]["W"].T, ((0, 2), (0, 0)))
    l0 = lps[0]["msg"]
    h, ph, qh, pvec = _node_init(
        x16, pos16, winT, params["lin_in"]["b"].reshape(1, D),
        l0["W1"][:, :D].T, l0["W1"][:, D:2 * D].T, l0["b1"].reshape(1, D))

    dst2d = jnp.pad(dst.reshape(1, E), ((0, 0), (0, EPAD - E)))
    src2d = jnp.pad(src.reshape(1, E), ((0, 0), (0, EPAD - E)))
    pvd, pvs = _sc_gather2(pvec, pvec, dst2d, src2d, D)
    feat = _edge_feat(pvd, pvs, ea16)

    for li, lp in enumerate(lps):
        mp = lp["msg"]
        wcdT = jnp.pad(mp["W1"][:, 2 * D:].T, ((0, 2), (0, 0)))
        ghd, ghs = _sc_gather2(ph, qh, dst2d, src2d, D)
        st1 = _stats1(ghd, ghs, feat, wcdT)
        ac1 = _affine(st1, mp["g1"], mp["be1"], float(E))
        t2, st2 = _t2_pass(ghd, ghs, feat, wcdT, ac1, mp["W2"].T)
        s2 = st2[0] + float(E) * mp["b2"]
        q2 = st2[1] + 2.0 * mp["b2"] * st2[0] + float(E) * mp["b2"] ** 2
        m2 = s2 / float(E)
        v2 = q2 / float(E) - m2 * m2
        a2 = mp["g2"] / jnp.sqrt(v2 + 1e-5)
        c2 = mp["be2"] - m2 * a2 + a2 * mp["b2"]
        ac2 = jnp.concatenate([a2.reshape(1, D), c2.reshape(1, D),
                               jnp.zeros((6, D), jnp.float32)], axis=0)
        msg = _msg_pass(t2, ac2)
        aggr2 = _sc_scatter_add(msg, dst)
        aggr = aggr2[0]
        aggr_b = aggr2[1]
        up = dict(lps[li]["upd"])
        up["W1b"] = up["W1"][:, D:].T
        up["W1"] = up["W1"][:, :D]
        if li + 1 < len(lps):
            nx = lps[li + 1]["msg"]
            h, ph, qh = _node_update(h, aggr, aggr_b, up, nx["W1"][:, :D].T,
                                     nx["W1"][:, D:2 * D].T,
                                     nx["b1"].reshape(1, D))
        else:
            wpT = jnp.pad(params["lin_pred"]["W"].T, ((0, 0), (0, 7)))
            bp = jnp.pad(params["lin_pred"]["b"].reshape(1, 1),
                         ((0, 0), (0, 7)))
            out = _final(h, aggr, aggr_b, up, batchf, wpT, bp)
    return out[:, 0].reshape(-1)


# speedup vs baseline: 3.4574x; 1.1997x over previous
import functools
import jax
from jax import lax
import jax.numpy as jnp
from jax.experimental import pallas as pl
from jax.experimental.pallas import tpu as pltpu
from jax.experimental.pallas import tpu_sc as plsc

N = 10000
E = 320000
D = 128
NG = 64
BE = 2560
GRID_E = E // BE
GW = 128
NC = 2
NS = 16
NCHUNK = E // GW
EPAD = 327680

_SC_MESH = plsc.VectorSubcoreMesh(core_axis_name="c", subcore_axis_name="s")


def _sc_gather2(tab_a, tab_b, idx_a2d, idx_b2d, width):
    @functools.partial(
        pl.kernel,
        out_type=(jax.ShapeDtypeStruct((EPAD, width), jnp.float32),
                  jax.ShapeDtypeStruct((EPAD, width), jnp.float32)),
        mesh=_SC_MESH,
        scratch_types=[pltpu.SemaphoreType.DMA, pltpu.SemaphoreType.DMA],
    )
    def k(a_hbm, b_hbm, ia_hbm, ib_hbm, oa_hbm, ob_hbm, sema, semb):
        def body(ia_v, ib_v, oa_v, ob_v):
            ca = pltpu.async_copy(a_hbm.at[ia_v.at[0]], oa_v, sema)
            cb = pltpu.async_copy(b_hbm.at[ib_v.at[0]], ob_v, semb)
            ca.wait()
            cb.wait()

        pltpu.emit_pipeline(
            body,
            grid=(EPAD // GW,),
            in_specs=[pl.BlockSpec((1, GW), lambda i: (0, i)),
                      pl.BlockSpec((1, GW), lambda i: (0, i))],
            out_specs=[pl.BlockSpec((GW, width), lambda i: (i, 0)),
                       pl.BlockSpec((GW, width), lambda i: (i, 0))],
            core_axis_name=("c", "s"),
            dimension_semantics=(pltpu.PARALLEL,),
        )(ia_hbm, ib_hbm, oa_hbm, ob_hbm)

    return k(tab_a, tab_b, idx_a2d, idx_b2d)


def _sc_scatter_add(msg, dst_flat):
    @functools.partial(
        pl.kernel,
        out_type=jax.ShapeDtypeStruct((NC, N, D), jnp.float32),
        mesh=_SC_MESH,
        scratch_types=[
            pltpu.VMEM((GW,), jnp.int32),
            pltpu.VMEM((GW, D), jnp.float32),
            pltpu.VMEM_SHARED((N, D), jnp.float32),
        ],
    )
    def k(msg_hbm, di_hbm, z_hbm, out_hbm, idx_v, rows_v, aggr_sh):
        cid = lax.axis_index("c")
        sid = lax.axis_index("s")

        @pl.loop(0, (N // 80 + NS - 1) // NS)
        def _(kk):
            row0 = (sid + kk * NS) * 80

            @pl.when(row0 < N)
            def _():
                pltpu.sync_copy(z_hbm.at[pl.ds(row0, 80)],
                                aggr_sh.at[pl.ds(row0, 80)])

        plsc.subcore_barrier()

        wid = cid * NS + sid

        @pl.loop(0, (NCHUNK + NC * NS - 1) // (NC * NS))
        def _(kk):
            c = wid + kk * NC * NS

            @pl.when(c < NCHUNK)
            def _():
                off = c * GW
                pltpu.sync_copy(di_hbm.at[pl.ds(off, GW)], idx_v)
                pltpu.sync_copy(msg_hbm.at[pl.ds(off, GW)], rows_v)
                pltpu.sync_copy(rows_v, aggr_sh.at[idx_v], add=True)

        plsc.subcore_barrier()

        @pl.when(sid == 0)
        def _():
            pltpu.sync_copy(aggr_sh, out_hbm.at[cid])

    return k(msg, dst_flat, jnp.zeros((N, D), jnp.float32))

_ACOS_C = (1.5707963050, -0.2145988016, 0.0889789874, -0.0501743046,
           0.0308918810, -0.0170881256, 0.0066700901, -0.0012624911)


def _acos(x):
    y = jnp.abs(x)
    p = _ACOS_C[7]
    for c in (_ACOS_C[6], _ACOS_C[5], _ACOS_C[4], _ACOS_C[3], _ACOS_C[2],
              _ACOS_C[1], _ACOS_C[0]):
        p = p * y + c
    r = jnp.sqrt(jnp.maximum(1.0 - y, 0.0)) * p
    return jnp.where(x < 0.0, jnp.pi - r, r)


def _init_kernel(x_ref, posp_ref, winT_ref, bin_ref, w1aT_ref, w1bT_ref,
                 b1_ref, h_ref, ph_ref, qh_ref, pvec_ref):
    h = jnp.dot(x_ref[...], winT_ref[...],
                preferred_element_type=jnp.float32) + bin_ref[...]
    h_ref[...] = h
    ph_ref[...] = jnp.dot(h, w1aT_ref[...], preferred_element_type=jnp.float32)
    qh_ref[...] = jnp.dot(h, w1bT_ref[...],
                          preferred_element_type=jnp.float32) + b1_ref[...]
    p = posp_ref[...]
    s = p[:, 0:2]
    e = p[:, 2:4]
    vec = e - s
    dist = jnp.sqrt(jnp.sum(vec * vec, axis=1, keepdims=True) + 1e-12)
    uv = vec / (dist + 1e-8)
    mid = (s + e) * 0.5
    z = jnp.zeros((p.shape[0], 123), jnp.float32)
    pvec_ref[...] = jnp.concatenate([dist, uv, mid, z], axis=1)


def _node_init(x16, pos16, winT, binb, w1aT, w1bT, b1):
    return pl.pallas_call(
        _init_kernel,
        out_shape=(
            jax.ShapeDtypeStruct((N, D), jnp.float32),
            jax.ShapeDtypeStruct((N, D), jnp.float32),
            jax.ShapeDtypeStruct((N, D), jnp.float32),
            jax.ShapeDtypeStruct((N, D), jnp.float32),
        ),
    )(x16, pos16, winT, binb, w1aT, w1bT, b1)


def _feat_kernel(pvd_ref, pvs_ref, ea_ref, feat_ref):
    pvd = pvd_ref[...]
    pvs = pvs_ref[...]
    d_i = pvd[:, 0:1]
    u_i = pvd[:, 1:3]
    m_i = pvd[:, 3:5]
    d_j = pvs[:, 0:1]
    u_j = pvs[:, 1:3]
    m_j = pvs[:, 3:5]
    dot = jnp.clip(jnp.sum(u_i * u_j, axis=1, keepdims=True),
                   -0.999999, 0.999999)
    angle = _acos(dot) * (180.0 / jnp.pi)
    cross = u_i[:, 0:1] * u_j[:, 1:2] - u_i[:, 1:2] * u_j[:, 0:1]
    is_lord = (cross > 0.0).astype(jnp.float32)
    diff = m_j - m_i
    spondy = jnp.sum(diff * u_i, axis=1, keepdims=True)
    perp = diff - spondy * u_i
    height = jnp.sqrt(jnp.sum(perp * perp, axis=1, keepdims=True) + 1e-12)
    z = jnp.zeros_like(d_i)
    feat_ref[...] = jnp.concatenate(
        [ea_ref[...][:, 0:4], d_i, u_i, d_j, u_j, angle, is_lord, spondy,
         height, z, z], axis=1)


def _edge_feat(pvd, pvs, ea16):
    spec16 = pl.BlockSpec((BE, 16), lambda i: (i, 0))
    specD = pl.BlockSpec((BE, D), lambda i: (i, 0))
    return pl.pallas_call(
        _feat_kernel,
        grid=(GRID_E,),
        in_specs=[specD, specD, spec16],
        out_specs=spec16,
        out_shape=jax.ShapeDtypeStruct((E, 16), jnp.float32),
    )(pvd, pvs, ea16)


def _stats1_kernel(ghd_ref, ghs_ref, feat_ref, wcdT_ref, acc_ref):
    t1 = (ghd_ref[...] + ghs_ref[...]
          + jnp.dot(feat_ref[...], wcdT_ref[...],
                    preferred_element_type=jnp.float32))
    i = pl.program_id(0)

    @pl.when(i == 0)
    def _():
        acc_ref[...] = jnp.zeros_like(acc_ref)

    acc_ref[0:1, :] += jnp.sum(t1, axis=0, keepdims=True)
    acc_ref[1:2, :] += jnp.sum(t1 * t1, axis=0, keepdims=True)


def _stats1(ghd, ghs, feat, wcdT):
    specD = pl.BlockSpec((BE, D), lambda i: (i, 0))
    return pl.pallas_call(
        _stats1_kernel,
        grid=(GRID_E,),
        in_specs=[specD, specD, pl.BlockSpec((BE, 16), lambda i: (i, 0)),
                  pl.BlockSpec((16, D), lambda i: (0, 0))],
        out_specs=pl.BlockSpec((8, D), lambda i: (0, 0)),
        out_shape=jax.ShapeDtypeStruct((8, D), jnp.float32),
    )(ghd, ghs, feat, wcdT)


def _t2_kernel(ghd_ref, ghs_ref, feat_ref, wcdT_ref, ac1_ref, w2T_ref,
               t2_ref, acc_ref):
    t1 = (ghd_ref[...] + ghs_ref[...]
          + jnp.dot(feat_ref[...], wcdT_ref[...],
                    preferred_element_type=jnp.float32))
    u = jnp.maximum(ac1_ref[0:1, :] * t1 + ac1_ref[1:2, :], 0.0)
    t2 = jnp.dot(u, w2T_ref[...], preferred_element_type=jnp.float32)
    t2_ref[...] = t2
    i = pl.program_id(0)

    @pl.when(i == 0)
    def _():
        acc_ref[...] = jnp.zeros_like(acc_ref)

    acc_ref[0:1, :] += jnp.sum(t2, axis=0, keepdims=True)
    acc_ref[1:2, :] += jnp.sum(t2 * t2, axis=0, keepdims=True)


def _t2_pass(ghd, ghs, feat, wcdT, ac1, w2T):
    specD = pl.BlockSpec((BE, D), lambda i: (i, 0))
    return pl.pallas_call(
        _t2_kernel,
        grid=(GRID_E,),
        in_specs=[specD, specD, pl.BlockSpec((BE, 16), lambda i: (i, 0)),
                  pl.BlockSpec((16, D), lambda i: (0, 0)),
                  pl.BlockSpec((8, D), lambda i: (0, 0)),
                  pl.BlockSpec((D, D), lambda i: (0, 0))],
        out_specs=(specD, pl.BlockSpec((8, D), lambda i: (0, 0))),
        out_shape=(jax.ShapeDtypeStruct((E, D), jnp.float32),
                   jax.ShapeDtypeStruct((8, D), jnp.float32)),
    )(ghd, ghs, feat, wcdT, ac1, w2T)


def _msg_kernel(t2_ref, ac2_ref, msg_ref):
    msg_ref[...] = jnp.maximum(
        ac2_ref[0:1, :] * t2_ref[...] + ac2_ref[1:2, :], 0.0)


def _msg_pass(t2, ac2):
    specD = pl.BlockSpec((BE, D), lambda i: (i, 0))
    return pl.pallas_call(
        _msg_kernel,
        grid=(GRID_E,),
        in_specs=[specD, pl.BlockSpec((8, D), lambda i: (0, 0))],
        out_specs=specD,
        out_shape=jax.ShapeDtypeStruct((E, D), jnp.float32),
    )(t2, ac2)


def _upd_kernel(h_ref, ag_ref, agb_ref, wu1aT_ref, wu1bT_ref, bu1_ref,
                g1_ref, be1_ref, wu2T_ref, bu2_ref, g2_ref, be2_ref,
                w1aT_ref, w1bT_ref, b1n_ref, hn_ref, ph_ref, qh_ref):
    h = h_ref[...]
    ag = ag_ref[...] + agb_ref[...]
    t = (jnp.dot(h, wu1aT_ref[...], preferred_element_type=jnp.float32)
         + jnp.dot(ag, wu1bT_ref[...],
                   preferred_element_type=jnp.float32) + bu1_ref[...])
    m = jnp.mean(t, axis=0, keepdims=True)
    v = jnp.mean(t * t, axis=0, keepdims=True) - m * m
    t = jnp.maximum(g1_ref[...] * (t - m) / jnp.sqrt(v + 1e-5)
                    + be1_ref[...], 0.0)
    t = jnp.dot(t, wu2T_ref[...], preferred_element_type=jnp.float32) \
        + bu2_ref[...]
    m = jnp.mean(t, axis=0, keepdims=True)
    v = jnp.mean(t * t, axis=0, keepdims=True) - m * m
    t = jnp.maximum(g2_ref[...] * (t - m) / jnp.sqrt(v + 1e-5)
                    + be2_ref[...], 0.0)
    hn = h + t
    hn_ref[...] = hn
    ph_ref[...] = jnp.dot(hn, w1aT_ref[...],
                          preferred_element_type=jnp.float32)
    qh_ref[...] = jnp.dot(hn, w1bT_ref[...],
                          preferred_element_type=jnp.float32) + b1n_ref[...]


def _node_update(h, aggr, aggr_b, up, w1aT_next, w1bT_next, b1_next):
    return pl.pallas_call(
        _upd_kernel,
        out_shape=(jax.ShapeDtypeStruct((N, D), jnp.float32),
                   jax.ShapeDtypeStruct((N, D), jnp.float32),
                   jax.ShapeDtypeStruct((N, D), jnp.float32)),
    )(h, aggr, aggr_b, up["W1"].T, up["W1b"], bu1 := up["b1"].reshape(1, D),
      up["g1"].reshape(1, D), up["be1"].reshape(1, D), up["W2"].T,
      up["b2"].reshape(1, D), up["g2"].reshape(1, D),
      up["be2"].reshape(1, D), w1aT_next, w1bT_next, b1_next)


def _final_kernel(h_ref, ag_ref, agb_ref, wu1aT_ref, wu1bT_ref, bu1_ref,
                  g1_ref, be1_ref, wu2T_ref, bu2_ref, g2_ref, be2_ref,
                  batch_ref, wpT_ref, bp_ref, out_ref):
    h = h_ref[...]
    ag = ag_ref[...] + agb_ref[...]
    t = (jnp.dot(h, wu1aT_ref[...], preferred_element_type=jnp.float32)
         + jnp.dot(ag, wu1bT_ref[...],
                   preferred_element_type=jnp.float32) + bu1_ref[...])
    m = jnp.mean(t, axis=0, keepdims=True)
    v = jnp.mean(t * t, axis=0, keepdims=True) - m * m
    t = jnp.maximum(g1_ref[...] * (t - m) / jnp.sqrt(v + 1e-5)
                    + be1_ref[...], 0.0)
    t = jnp.dot(t, wu2T_ref[...], preferred_element_type=jnp.float32) \
        + bu2_ref[...]
    m = jnp.mean(t, axis=0, keepdims=True)
    v = jnp.mean(t * t, axis=0, keepdims=True) - m * m
    t = jnp.maximum(g2_ref[...] * (t - m) / jnp.sqrt(v + 1e-5)
                    + be2_ref[...], 0.0)
    hn = h + t
    b = batch_ref[...][:, 0:1]
    gid = jax.lax.broadcasted_iota(jnp.int32, (1, NG), 1)
    onehot = (b == gid).astype(jnp.float32)
    sums = jax.lax.dot_general(onehot, hn, (((0,), (0,)), ((), ())),
                               preferred_element_type=jnp.float32)
    cnt8 = jax.lax.dot_general(onehot, jnp.ones((h.shape[0], 8), jnp.float32),
                               (((0,), (0,)), ((), ())),
                               preferred_element_type=jnp.float32)
    hg = sums / jnp.maximum(cnt8[:, 0:1], 1.0)
    out_ref[...] = jnp.dot(hg, wpT_ref[...],
                           preferred_element_type=jnp.float32) + bp_ref[...]


def _final(h, aggr, aggr_b, up, batchf, wpT, bp):
    return pl.pallas_call(
        _final_kernel,
        out_shape=jax.ShapeDtypeStruct((NG, 8), jnp.float32),
    )(h, aggr, aggr_b, up["W1"].T, up["W1b"], up["b1"].reshape(1, D),
      up["g1"].reshape(1, D), up["be1"].reshape(1, D), up["W2"].T,
      up["b2"].reshape(1, D), up["g2"].reshape(1, D),
      up["be2"].reshape(1, D), batchf, wpT, bp)


def _affine(stats, g, be, n):
    m = stats[0] / n
    v = stats[1] / n - m * m
    a = g / jnp.sqrt(v + 1e-5)
    c = be - m * a
    return jnp.concatenate([a.reshape(1, D), c.reshape(1, D),
                            jnp.zeros((6, D), jnp.float32)], axis=0)


def kernel(x, pos, edge_index, edge_attr, batch, params):
    src = edge_index[0].astype(jnp.int32)
    dst = edge_index[1].astype(jnp.int32)
    x16 = jnp.pad(x, ((0, 0), (0, 2)))
    pos16 = jnp.pad(pos, ((0, 0), (0, 12)))
    ea16 = jnp.pad(edge_attr, ((0, 0), (0, 12)))
    batchf = jnp.pad(batch.astype(jnp.int32).reshape(N, 1),
                     ((0, 0), (0, 7)))

    lps = params["layers"]
    winT = jnp.pad(params["lin_in"]["W"].T, ((0, 2), (0, 0)))
    l0 = lps[0]["msg"]
    h, ph, qh, pvec = _node_init(
        x16, pos16, winT, params["lin_in"]["b"].reshape(1, D),
        l0["W1"][:, :D].T, l0["W1"][:, D:2 * D].T, l0["b1"].reshape(1, D))

    dst2d = jnp.pad(dst.reshape(1, E), ((0, 0), (0, EPAD - E)))
    src2d = jnp.pad(src.reshape(1, E), ((0, 0), (0, EPAD - E)))
    pvd, pvs = _sc_gather2(pvec, pvec, dst2d, src2d, D)
    feat = _edge_feat(pvd, pvs, ea16)

    for li, lp in enumerate(lps):
        mp = lp["msg"]
        wcdT = jnp.pad(mp["W1"][:, 2 * D:].T, ((0, 2), (0, 0)))
        ghd, ghs = _sc_gather2(ph, qh, dst2d, src2d, D)
        st1 = _stats1(ghd, ghs, feat, wcdT)
        ac1 = _affine(st1, mp["g1"], mp["be1"], float(E))
        t2, st2 = _t2_pass(ghd, ghs, feat, wcdT, ac1, mp["W2"].T)
        s2 = st2[0] + float(E) * mp["b2"]
        q2 = st2[1] + 2.0 * mp["b2"] * st2[0] + float(E) * mp["b2"] ** 2
        m2 = s2 / float(E)
        v2 = q2 / float(E) - m2 * m2
        a2 = mp["g2"] / jnp.sqrt(v2 + 1e-5)
        c2 = mp["be2"] - m2 * a2 + a2 * mp["b2"]
        ac2 = jnp.concatenate([a2.reshape(1, D), c2.reshape(1, D),
                               jnp.zeros((6, D), jnp.float32)], axis=0)
        msg = _msg_pass(t2, ac2)
        aggr2 = _sc_scatter_add(msg, dst)
        aggr = aggr2[0]
        aggr_b = aggr2[1]
        up = dict(lps[li]["upd"])
        up["W1b"] = up["W1"][:, D:].T
        up["W1"] = up["W1"][:, :D]
        if li + 1 < len(lps):
            nx = lps[li + 1]["msg"]
            h, ph, qh = _node_update(h, aggr, aggr_b, up, nx["W1"][:, :D].T,
                                     nx["W1"][:, D:2 * D].T,
                                     nx["b1"].reshape(1, D))
        else:
            wpT = jnp.pad(params["lin_pred"]["W"].T, ((0, 0), (0, 7)))
            bp = jnp.pad(params["lin_pred"]["b"].reshape(1, 1),
                         ((0, 0), (0, 7)))
            out = _final(h, aggr, aggr_b, up, batchf, wpT, bp)
    return out[:, 0].reshape(-1)


# speedup vs baseline: 3.5240x; 1.0193x over previous
import functools
import jax
from jax import lax
import jax.numpy as jnp
from jax.experimental import pallas as pl
from jax.experimental.pallas import tpu as pltpu
from jax.experimental.pallas import tpu_sc as plsc

N = 10000
E = 320000
D = 128
NG = 64
BE = 2560
GRID_E = E // BE
GW = 128
NC = 2
NS = 16
NCHUNK = E // GW
EPAD = 327680

_SC_MESH = plsc.VectorSubcoreMesh(core_axis_name="c", subcore_axis_name="s")


def _sc_gather2(tab_a, tab_b, idx_a2d, idx_b2d, width):
    @functools.partial(
        pl.kernel,
        out_type=(jax.ShapeDtypeStruct((EPAD, width), jnp.float32),
                  jax.ShapeDtypeStruct((EPAD, width), jnp.float32)),
        mesh=_SC_MESH,
        scratch_types=[pltpu.SemaphoreType.DMA, pltpu.SemaphoreType.DMA],
    )
    def k(a_hbm, b_hbm, ia_hbm, ib_hbm, oa_hbm, ob_hbm, sema, semb):
        def body(ia_v, ib_v, oa_v, ob_v):
            ca = pltpu.async_copy(a_hbm.at[ia_v.at[0]], oa_v, sema)
            cb = pltpu.async_copy(b_hbm.at[ib_v.at[0]], ob_v, semb)
            ca.wait()
            cb.wait()

        pltpu.emit_pipeline(
            body,
            grid=(EPAD // GW,),
            in_specs=[pl.BlockSpec((1, GW), lambda i: (0, i)),
                      pl.BlockSpec((1, GW), lambda i: (0, i))],
            out_specs=[pl.BlockSpec((GW, width), lambda i: (i, 0)),
                       pl.BlockSpec((GW, width), lambda i: (i, 0))],
            core_axis_name=("c", "s"),
            dimension_semantics=(pltpu.PARALLEL,),
        )(ia_hbm, ib_hbm, oa_hbm, ob_hbm)

    return k(tab_a, tab_b, idx_a2d, idx_b2d)


def _sc_scatter_add(msg, dst_flat):
    @functools.partial(
        pl.kernel,
        out_type=jax.ShapeDtypeStruct((NC, N, D), jnp.float32),
        mesh=_SC_MESH,
        scratch_types=[
            pltpu.VMEM((GW,), jnp.int32),
            pltpu.VMEM((GW, D), jnp.float32),
            pltpu.VMEM_SHARED((N, D), jnp.float32),
            pltpu.SemaphoreType.DMA,
            pltpu.SemaphoreType.DMA,
        ],
    )
    def k(msg_hbm, di_hbm, z_hbm, out_hbm, idx_v, rows_v, aggr_sh,
          sem_i, sem_r):
        cid = lax.axis_index("c")
        sid = lax.axis_index("s")

        @pl.loop(0, (N // 80 + NS - 1) // NS)
        def _(kk):
            row0 = (sid + kk * NS) * 80

            @pl.when(row0 < N)
            def _():
                pltpu.sync_copy(z_hbm.at[pl.ds(row0, 80)],
                                aggr_sh.at[pl.ds(row0, 80)])

        plsc.subcore_barrier()

        wid = cid * NS + sid

        @pl.loop(0, (NCHUNK + NC * NS - 1) // (NC * NS))
        def _(kk):
            c = wid + kk * NC * NS

            @pl.when(c < NCHUNK)
            def _():
                off = c * GW
                ci = pltpu.async_copy(di_hbm.at[pl.ds(off, GW)], idx_v,
                                      sem_i)
                cr = pltpu.async_copy(msg_hbm.at[pl.ds(off, GW)], rows_v,
                                      sem_r)
                ci.wait()
                cr.wait()
                pltpu.sync_copy(rows_v, aggr_sh.at[idx_v], add=True)

        plsc.subcore_barrier()

        @pl.when(sid == 0)
        def _():
            pltpu.sync_copy(aggr_sh, out_hbm.at[cid])

    return k(msg, dst_flat, jnp.zeros((N, D), jnp.float32))

_ACOS_C = (1.5707963050, -0.2145988016, 0.0889789874, -0.0501743046,
           0.0308918810, -0.0170881256, 0.0066700901, -0.0012624911)


def _acos(x):
    y = jnp.abs(x)
    p = _ACOS_C[7]
    for c in (_ACOS_C[6], _ACOS_C[5], _ACOS_C[4], _ACOS_C[3], _ACOS_C[2],
              _ACOS_C[1], _ACOS_C[0]):
        p = p * y + c
    r = jnp.sqrt(jnp.maximum(1.0 - y, 0.0)) * p
    return jnp.where(x < 0.0, jnp.pi - r, r)


def _init_kernel(x_ref, posp_ref, winT_ref, bin_ref, w1aT_ref, w1bT_ref,
                 b1_ref, h_ref, ph_ref, qh_ref, pvec_ref):
    h = jnp.dot(x_ref[...], winT_ref[...],
                preferred_element_type=jnp.float32) + bin_ref[...]
    h_ref[...] = h
    ph_ref[...] = jnp.dot(h, w1aT_ref[...], preferred_element_type=jnp.float32)
    qh_ref[...] = jnp.dot(h, w1bT_ref[...],
                          preferred_element_type=jnp.float32) + b1_ref[...]
    p = posp_ref[...]
    s = p[:, 0:2]
    e = p[:, 2:4]
    vec = e - s
    dist = jnp.sqrt(jnp.sum(vec * vec, axis=1, keepdims=True) + 1e-12)
    uv = vec / (dist + 1e-8)
    mid = (s + e) * 0.5
    z = jnp.zeros((p.shape[0], 123), jnp.float32)
    pvec_ref[...] = jnp.concatenate([dist, uv, mid, z], axis=1)


def _node_init(x16, pos16, winT, binb, w1aT, w1bT, b1):
    return pl.pallas_call(
        _init_kernel,
        out_shape=(
            jax.ShapeDtypeStruct((N, D), jnp.float32),
            jax.ShapeDtypeStruct((N, D), jnp.float32),
            jax.ShapeDtypeStruct((N, D), jnp.float32),
            jax.ShapeDtypeStruct((N, D), jnp.float32),
        ),
    )(x16, pos16, winT, binb, w1aT, w1bT, b1)


def _feat_kernel(pvd_ref, pvs_ref, ea_ref, feat_ref):
    pvd = pvd_ref[...]
    pvs = pvs_ref[...]
    d_i = pvd[:, 0:1]
    u_i = pvd[:, 1:3]
    m_i = pvd[:, 3:5]
    d_j = pvs[:, 0:1]
    u_j = pvs[:, 1:3]
    m_j = pvs[:, 3:5]
    dot = jnp.clip(jnp.sum(u_i * u_j, axis=1, keepdims=True),
                   -0.999999, 0.999999)
    angle = _acos(dot) * (180.0 / jnp.pi)
    cross = u_i[:, 0:1] * u_j[:, 1:2] - u_i[:, 1:2] * u_j[:, 0:1]
    is_lord = (cross > 0.0).astype(jnp.float32)
    diff = m_j - m_i
    spondy = jnp.sum(diff * u_i, axis=1, keepdims=True)
    perp = diff - spondy * u_i
    height = jnp.sqrt(jnp.sum(perp * perp, axis=1, keepdims=True) + 1e-12)
    z = jnp.zeros_like(d_i)
    feat_ref[...] = jnp.concatenate(
        [ea_ref[...][:, 0:4], d_i, u_i, d_j, u_j, angle, is_lord, spondy,
         height, z, z], axis=1)


def _edge_feat(pvd, pvs, ea16):
    spec16 = pl.BlockSpec((BE, 16), lambda i: (i, 0))
    specD = pl.BlockSpec((BE, D), lambda i: (i, 0))
    return pl.pallas_call(
        _feat_kernel,
        grid=(GRID_E,),
        in_specs=[specD, specD, spec16],
        out_specs=spec16,
        out_shape=jax.ShapeDtypeStruct((E, 16), jnp.float32),
    )(pvd, pvs, ea16)


def _stats1_kernel(ghd_ref, ghs_ref, feat_ref, wcdT_ref, acc_ref):
    t1 = (ghd_ref[...] + ghs_ref[...]
          + jnp.dot(feat_ref[...], wcdT_ref[...],
                    preferred_element_type=jnp.float32))
    i = pl.program_id(0)

    @pl.when(i == 0)
    def _():
        acc_ref[...] = jnp.zeros_like(acc_ref)

    acc_ref[0:1, :] += jnp.sum(t1, axis=0, keepdims=True)
    acc_ref[1:2, :] += jnp.sum(t1 * t1, axis=0, keepdims=True)


def _stats1(ghd, ghs, feat, wcdT):
    specD = pl.BlockSpec((BE, D), lambda i: (i, 0))
    return pl.pallas_call(
        _stats1_kernel,
        grid=(GRID_E,),
        in_specs=[specD, specD, pl.BlockSpec((BE, 16), lambda i: (i, 0)),
                  pl.BlockSpec((16, D), lambda i: (0, 0))],
        out_specs=pl.BlockSpec((8, D), lambda i: (0, 0)),
        out_shape=jax.ShapeDtypeStruct((8, D), jnp.float32),
    )(ghd, ghs, feat, wcdT)


def _t2_kernel(ghd_ref, ghs_ref, feat_ref, wcdT_ref, ac1_ref, w2T_ref,
               t2_ref, acc_ref):
    t1 = (ghd_ref[...] + ghs_ref[...]
          + jnp.dot(feat_ref[...], wcdT_ref[...],
                    preferred_element_type=jnp.float32))
    u = jnp.maximum(ac1_ref[0:1, :] * t1 + ac1_ref[1:2, :], 0.0)
    t2 = jnp.dot(u, w2T_ref[...], preferred_element_type=jnp.float32)
    t2_ref[...] = t2
    i = pl.program_id(0)

    @pl.when(i == 0)
    def _():
        acc_ref[...] = jnp.zeros_like(acc_ref)

    acc_ref[0:1, :] += jnp.sum(t2, axis=0, keepdims=True)
    acc_ref[1:2, :] += jnp.sum(t2 * t2, axis=0, keepdims=True)


def _t2_pass(ghd, ghs, feat, wcdT, ac1, w2T):
    specD = pl.BlockSpec((BE, D), lambda i: (i, 0))
    return pl.pallas_call(
        _t2_kernel,
        grid=(GRID_E,),
        in_specs=[specD, specD, pl.BlockSpec((BE, 16), lambda i: (i, 0)),
                  pl.BlockSpec((16, D), lambda i: (0, 0)),
                  pl.BlockSpec((8, D), lambda i: (0, 0)),
                  pl.BlockSpec((D, D), lambda i: (0, 0))],
        out_specs=(specD, pl.BlockSpec((8, D), lambda i: (0, 0))),
        out_shape=(jax.ShapeDtypeStruct((E, D), jnp.float32),
                   jax.ShapeDtypeStruct((8, D), jnp.float32)),
    )(ghd, ghs, feat, wcdT, ac1, w2T)


def _msg_kernel(t2_ref, ac2_ref, msg_ref):
    msg_ref[...] = jnp.maximum(
        ac2_ref[0:1, :] * t2_ref[...] + ac2_ref[1:2, :], 0.0)


def _msg_pass(t2, ac2):
    specD = pl.BlockSpec((BE, D), lambda i: (i, 0))
    return pl.pallas_call(
        _msg_kernel,
        grid=(GRID_E,),
        in_specs=[specD, pl.BlockSpec((8, D), lambda i: (0, 0))],
        out_specs=specD,
        out_shape=jax.ShapeDtypeStruct((E, D), jnp.float32),
    )(t2, ac2)


def _upd_kernel(h_ref, ag_ref, agb_ref, wu1aT_ref, wu1bT_ref, bu1_ref,
                g1_ref, be1_ref, wu2T_ref, bu2_ref, g2_ref, be2_ref,
                w1aT_ref, w1bT_ref, b1n_ref, hn_ref, ph_ref, qh_ref):
    h = h_ref[...]
    ag = ag_ref[...] + agb_ref[...]
    t = (jnp.dot(h, wu1aT_ref[...], preferred_element_type=jnp.float32)
         + jnp.dot(ag, wu1bT_ref[...],
                   preferred_element_type=jnp.float32) + bu1_ref[...])
    m = jnp.mean(t, axis=0, keepdims=True)
    v = jnp.mean(t * t, axis=0, keepdims=True) - m * m
    t = jnp.maximum(g1_ref[...] * (t - m) / jnp.sqrt(v + 1e-5)
                    + be1_ref[...], 0.0)
    t = jnp.dot(t, wu2T_ref[...], preferred_element_type=jnp.float32) \
        + bu2_ref[...]
    m = jnp.mean(t, axis=0, keepdims=True)
    v = jnp.mean(t * t, axis=0, keepdims=True) - m * m
    t = jnp.maximum(g2_ref[...] * (t - m) / jnp.sqrt(v + 1e-5)
                    + be2_ref[...], 0.0)
    hn = h + t
    hn_ref[...] = hn
    ph_ref[...] = jnp.dot(hn, w1aT_ref[...],
                          preferred_element_type=jnp.float32)
    qh_ref[...] = jnp.dot(hn, w1bT_ref[...],
                          preferred_element_type=jnp.float32) + b1n_ref[...]


def _node_update(h, aggr, aggr_b, up, w1aT_next, w1bT_next, b1_next):
    return pl.pallas_call(
        _upd_kernel,
        out_shape=(jax.ShapeDtypeStruct((N, D), jnp.float32),
                   jax.ShapeDtypeStruct((N, D), jnp.float32),
                   jax.ShapeDtypeStruct((N, D), jnp.float32)),
    )(h, aggr, aggr_b, up["W1"].T, up["W1b"], bu1 := up["b1"].reshape(1, D),
      up["g1"].reshape(1, D), up["be1"].reshape(1, D), up["W2"].T,
      up["b2"].reshape(1, D), up["g2"].reshape(1, D),
      up["be2"].reshape(1, D), w1aT_next, w1bT_next, b1_next)


def _final_kernel(h_ref, ag_ref, agb_ref, wu1aT_ref, wu1bT_ref, bu1_ref,
                  g1_ref, be1_ref, wu2T_ref, bu2_ref, g2_ref, be2_ref,
                  batch_ref, wpT_ref, bp_ref, out_ref):
    h = h_ref[...]
    ag = ag_ref[...] + agb_ref[...]
    t = (jnp.dot(h, wu1aT_ref[...], preferred_element_type=jnp.float32)
         + jnp.dot(ag, wu1bT_ref[...],
                   preferred_element_type=jnp.float32) + bu1_ref[...])
    m = jnp.mean(t, axis=0, keepdims=True)
    v = jnp.mean(t * t, axis=0, keepdims=True) - m * m
    t = jnp.maximum(g1_ref[...] * (t - m) / jnp.sqrt(v + 1e-5)
                    + be1_ref[...], 0.0)
    t = jnp.dot(t, wu2T_ref[...], preferred_element_type=jnp.float32) \
        + bu2_ref[...]
    m = jnp.mean(t, axis=0, keepdims=True)
    v = jnp.mean(t * t, axis=0, keepdims=True) - m * m
    t = jnp.maximum(g2_ref[...] * (t - m) / jnp.sqrt(v + 1e-5)
                    + be2_ref[...], 0.0)
    hn = h + t
    b = batch_ref[...][:, 0:1]
    gid = jax.lax.broadcasted_iota(jnp.int32, (1, NG), 1)
    onehot = (b == gid).astype(jnp.float32)
    sums = jax.lax.dot_general(onehot, hn, (((0,), (0,)), ((), ())),
                               preferred_element_type=jnp.float32)
    cnt8 = jax.lax.dot_general(onehot, jnp.ones((h.shape[0], 8), jnp.float32),
                               (((0,), (0,)), ((), ())),
                               preferred_element_type=jnp.float32)
    hg = sums / jnp.maximum(cnt8[:, 0:1], 1.0)
    out_ref[...] = jnp.dot(hg, wpT_ref[...],
                           preferred_element_type=jnp.float32) + bp_ref[...]


def _final(h, aggr, aggr_b, up, batchf, wpT, bp):
    return pl.pallas_call(
        _final_kernel,
        out_shape=jax.ShapeDtypeStruct((NG, 8), jnp.float32),
    )(h, aggr, aggr_b, up["W1"].T, up["W1b"], up["b1"].reshape(1, D),
      up["g1"].reshape(1, D), up["be1"].reshape(1, D), up["W2"].T,
      up["b2"].reshape(1, D), up["g2"].reshape(1, D),
      up["be2"].reshape(1, D), batchf, wpT, bp)


def _affine(stats, g, be, n):
    m = stats[0] / n
    v = stats[1] / n - m * m
    a = g / jnp.sqrt(v + 1e-5)
    c = be - m * a
    return jnp.concatenate([a.reshape(1, D), c.reshape(1, D),
                            jnp.zeros((6, D), jnp.float32)], axis=0)


def kernel(x, pos, edge_index, edge_attr, batch, params):
    src = edge_index[0].astype(jnp.int32)
    dst = edge_index[1].astype(jnp.int32)
    x16 = jnp.pad(x, ((0, 0), (0, 2)))
    pos16 = jnp.pad(pos, ((0, 0), (0, 12)))
    ea16 = jnp.pad(edge_attr, ((0, 0), (0, 12)))
    batchf = jnp.pad(batch.astype(jnp.int32).reshape(N, 1),
                     ((0, 0), (0, 7)))

    lps = params["layers"]
    winT = jnp.pad(params["lin_in"]["W"].T, ((0, 2), (0, 0)))
    l0 = lps[0]["msg"]
    h, ph, qh, pvec = _node_init(
        x16, pos16, winT, params["lin_in"]["b"].reshape(1, D),
        l0["W1"][:, :D].T, l0["W1"][:, D:2 * D].T, l0["b1"].reshape(1, D))

    dst2d = jnp.pad(dst.reshape(1, E), ((0, 0), (0, EPAD - E)))
    src2d = jnp.pad(src.reshape(1, E), ((0, 0), (0, EPAD - E)))
    pvd, pvs = _sc_gather2(pvec, pvec, dst2d, src2d, D)
    feat = _edge_feat(pvd, pvs, ea16)

    for li, lp in enumerate(lps):
        mp = lp["msg"]
        wcdT = jnp.pad(mp["W1"][:, 2 * D:].T, ((0, 2), (0, 0)))
        ghd, ghs = _sc_gather2(ph, qh, dst2d, src2d, D)
        st1 = _stats1(ghd, ghs, feat, wcdT)
        ac1 = _affine(st1, mp["g1"], mp["be1"], float(E))
        t2, st2 = _t2_pass(ghd, ghs, feat, wcdT, ac1, mp["W2"].T)
        s2 = st2[0] + float(E) * mp["b2"]
        q2 = st2[1] + 2.0 * mp["b2"] * st2[0] + float(E) * mp["b2"] ** 2
        m2 = s2 / float(E)
        v2 = q2 / float(E) - m2 * m2
        a2 = mp["g2"] / jnp.sqrt(v2 + 1e-5)
        c2 = mp["be2"] - m2 * a2 + a2 * mp["b2"]
        ac2 = jnp.concatenate([a2.reshape(1, D), c2.reshape(1, D),
                               jnp.zeros((6, D), jnp.float32)], axis=0)
        msg = _msg_pass(t2, ac2)
        aggr2 = _sc_scatter_add(msg, dst)
        aggr = aggr2[0]
        aggr_b = aggr2[1]
        up = dict(lps[li]["upd"])
        up["W1b"] = up["W1"][:, D:].T
        up["W1"] = up["W1"][:, :D]
        if li + 1 < len(lps):
            nx = lps[li + 1]["msg"]
            h, ph, qh = _node_update(h, aggr, aggr_b, up, nx["W1"][:, :D].T,
                                     nx["W1"][:, D:2 * D].T,
                                     nx["b1"].reshape(1, D))
        else:
            wpT = jnp.pad(params["lin_pred"]["W"].T, ((0, 0), (0, 7)))
            bp = jnp.pad(params["lin_pred"]["b"].reshape(1, 1),
                         ((0, 0), (0, 7)))
            out = _final(h, aggr, aggr_b, up, batchf, wpT, bp)
    return out[:, 0].reshape(-1)
